# Initial kernel scaffold; baseline (speedup 1.0000x reference)
import jax
import jax.numpy as jnp
from jax.experimental import pallas as pl


def kernel(x_nodes, edge_index, degrees, identifiers, edge_features, batch, atom_emb, id_emb, bond_emb, W1, b1, W2, b2, proj_W, proj_b):
    raise NotImplementedError("write your pallas kernel here")


# speedup vs baseline: 34.4234x; 34.4234x over previous
import functools

import jax
import jax.numpy as jnp
from jax import lax
from jax.experimental import pallas as pl
from jax.experimental.pallas import tpu as pltpu
from jax.experimental.pallas import tpu_sc as plsc

N_NODES = 10000
N_EDGES = 320000
EMB = 128
N_LAYERS = 5
N_GRAPHS = 64
OUT = 128
ID_VOCAB = 50

NC = 2
NS = 16
NW = NC * NS
CHUNK = 128
N_CHUNKS = N_EDGES // CHUNK
ROWS_PER_TILE = 624
TAIL_ROWS = N_NODES - NS * ROWS_PER_TILE

F32 = jnp.float32
HI = lax.Precision.HIGHEST


def _sc_mesh():
    return plsc.VectorSubcoreMesh(
        core_axis_name="c", subcore_axis_name="s", num_cores=NC, num_subcores=NS)


def _zero_vmem_block(buf):
    zero = jnp.zeros((16,), F32)

    def row(i, _):
        for k in range(EMB // 16):
            buf[i, pl.ds(k * 16, 16)] = zero
        return 0

    lax.fori_loop(0, CHUNK, row, 0)


def _zero_agg_stripe(buf, agg, s):
    stripe = s * ROWS_PER_TILE
    n_full = ROWS_PER_TILE // CHUNK
    rem = ROWS_PER_TILE - n_full * CHUNK
    for g in range(n_full):
        pltpu.sync_copy(buf, agg.at[pl.ds(stripe + g * CHUNK, CHUNK)])
    if rem:
        pltpu.sync_copy(buf.at[pl.ds(0, rem)],
                        agg.at[pl.ds(stripe + n_full * CHUNK, rem)])

    @pl.when(s == NS - 1)
    def _():
        pltpu.sync_copy(buf.at[pl.ds(0, TAIL_ROWS)],
                        agg.at[pl.ds(NS * ROWS_PER_TILE, TAIL_ROWS)])


def _copy_out_stripe(agg, out_hbm, c, s):
    stripe = s * ROWS_PER_TILE
    pltpu.sync_copy(agg.at[pl.ds(stripe, ROWS_PER_TILE)],
                    out_hbm.at[c, pl.ds(stripe, ROWS_PER_TILE)])

    @pl.when(s == NS - 1)
    def _():
        pltpu.sync_copy(agg.at[pl.ds(NS * ROWS_PER_TILE, TAIL_ROWS)],
                        out_hbm.at[c, pl.ds(NS * ROWS_PER_TILE, TAIL_ROWS)])


def _edge_layer0_call():
    @functools.partial(
        pl.kernel,
        out_type=jax.ShapeDtypeStruct((NC, N_NODES, EMB), F32),
        mesh=_sc_mesh(),
        scratch_types=[
            pltpu.VMEM((CHUNK,), jnp.int32),
            pltpu.VMEM((CHUNK,), jnp.int32),
            pltpu.VMEM((CHUNK,), jnp.int32),
            pltpu.VMEM((CHUNK, EMB), F32),
            pltpu.VMEM((CHUNK, EMB), F32),
            pltpu.VMEM_SHARED((N_NODES, EMB), F32),
            pltpu.SemaphoreType.DMA,
        ],
    )
    def k(x_hbm, t_hbm, src_hbm, cidx_hbm, dst_hbm, out_hbm,
          src_v, cidx_v, dst_v, xrows, trows, agg, sem):
        c = lax.axis_index("c")
        s = lax.axis_index("s")
        w = c * NS + s

        _zero_vmem_block(xrows)
        _zero_agg_stripe(xrows, agg, s)
        plsc.subcore_barrier()

        n_my = 78 + jnp.where(w < N_CHUNKS - 78 * NW, 1, 0)

        def body(g, _):
            base = (w + g * NW) * CHUNK
            pltpu.sync_copy(src_hbm.at[pl.ds(base, CHUNK)], src_v)
            pltpu.sync_copy(cidx_hbm.at[pl.ds(base, CHUNK)], cidx_v)
            pltpu.sync_copy(dst_hbm.at[pl.ds(base, CHUNK)], dst_v)
            pltpu.async_copy(x_hbm.at[src_v], xrows, sem).wait()
            pltpu.async_copy(t_hbm.at[cidx_v], trows, sem).wait()

            def row(i, _):
                for kk in range(EMB // 16):
                    sl = pl.ds(kk * 16, 16)
                    v = xrows[i, sl] + trows[i, sl]
                    xrows[i, sl] = jnp.maximum(v, 0.0)
                return 0

            lax.fori_loop(0, CHUNK, row, 0)
            pltpu.sync_copy(xrows, agg.at[dst_v], add=True)
            return 0

        lax.fori_loop(0, n_my, body, 0)
        plsc.subcore_barrier()
        _copy_out_stripe(agg, out_hbm, c, s)

    return k


def _edge_gather_scatter_call():
    @functools.partial(
        pl.kernel,
        out_type=jax.ShapeDtypeStruct((NC, N_NODES, EMB), F32),
        mesh=_sc_mesh(),
        scratch_types=[
            pltpu.VMEM((CHUNK,), jnp.int32),
            pltpu.VMEM((CHUNK,), jnp.int32),
            pltpu.VMEM((CHUNK, EMB), F32),
            pltpu.VMEM_SHARED((N_NODES, EMB), F32),
            pltpu.SemaphoreType.DMA,
        ],
    )
    def k(y_hbm, yidx_hbm, dst_hbm, out_hbm, yidx_v, dst_v, rows, agg, sem):
        c = lax.axis_index("c")
        s = lax.axis_index("s")
        w = c * NS + s

        _zero_vmem_block(rows)
        _zero_agg_stripe(rows, agg, s)
        plsc.subcore_barrier()

        n_my = 78 + jnp.where(w < N_CHUNKS - 78 * NW, 1, 0)

        def body(g, _):
            base = (w + g * NW) * CHUNK
            pltpu.sync_copy(yidx_hbm.at[pl.ds(base, CHUNK)], yidx_v)
            pltpu.sync_copy(dst_hbm.at[pl.ds(base, CHUNK)], dst_v)
            pltpu.async_copy(y_hbm.at[yidx_v], rows, sem).wait()
            pltpu.sync_copy(rows, agg.at[dst_v], add=True)
            return 0

        lax.fori_loop(0, n_my, body, 0)
        plsc.subcore_barrier()
        _copy_out_stripe(agg, out_hbm, c, s)

    return k


ROW_BLK = 1000
N_BLKS = N_NODES // ROW_BLK


def _h_kernel(xn_pad, d_pad, base):
    def body(xn_ref, d_ref, b_ref, o_ref):
        o_ref[...] = (
            jnp.dot(xn_ref[...], d_ref[...], preferred_element_type=F32,
                    precision=HI)
            + b_ref[...])

    return pl.pallas_call(
        body,
        grid=(N_BLKS,),
        in_specs=[
            pl.BlockSpec((ROW_BLK, 16), lambda i: (i, 0)),
            pl.BlockSpec((16, EMB), lambda i: (0, 0)),
            pl.BlockSpec((1, EMB), lambda i: (0, 0)),
        ],
        out_specs=pl.BlockSpec((ROW_BLK, EMB), lambda i: (i, 0)),
        out_shape=jax.ShapeDtypeStruct((N_NODES, EMB), F32),
    )(xn_pad, d_pad, base)


def _mlp_kernel(x, agg, W1l, b1l, W2l, b2l, t_next, relu_out, emit_y):
    def body(x_ref, a_ref, w1_ref, b1_ref, w2_ref, b2_ref, t_ref,
             xo_ref, yo_ref=None):
        u = x_ref[...] + a_ref[0] + a_ref[1]
        h1 = jnp.maximum(
            jnp.dot(u, w1_ref[...], preferred_element_type=F32, precision=HI)
            + b1_ref[...], 0.0)
        o = (jnp.dot(h1, w2_ref[...], preferred_element_type=F32, precision=HI)
             + b2_ref[...])
        if relu_out:
            o = jnp.maximum(o, 0.0)
        xo_ref[...] = o
        if emit_y:
            for cc in range(8):
                yo_ref[cc] = jnp.maximum(o + t_ref[cc], 0.0)

    out_shapes = [jax.ShapeDtypeStruct((N_NODES, EMB), F32)]
    out_specs = [pl.BlockSpec((ROW_BLK, EMB), lambda i: (i, 0))]
    if emit_y:
        out_shapes.append(jax.ShapeDtypeStruct((8, N_NODES, EMB), F32))
        out_specs.append(pl.BlockSpec((8, ROW_BLK, EMB), lambda i: (0, i, 0)))

    if emit_y:
        wrapped = body
    else:
        def wrapped(x_ref, a_ref, w1_ref, b1_ref, w2_ref, b2_ref, t_ref,
                    xo_ref):
            body(x_ref, a_ref, w1_ref, b1_ref, w2_ref, b2_ref, t_ref, xo_ref)

    res = pl.pallas_call(
        wrapped,
        grid=(N_BLKS,),
        in_specs=[
            pl.BlockSpec((ROW_BLK, EMB), lambda i: (i, 0)),
            pl.BlockSpec((NC, ROW_BLK, EMB), lambda i: (0, i, 0)),
            pl.BlockSpec((EMB, 2 * EMB), lambda i: (0, 0)),
            pl.BlockSpec((1, 2 * EMB), lambda i: (0, 0)),
            pl.BlockSpec((2 * EMB, EMB), lambda i: (0, 0)),
            pl.BlockSpec((1, EMB), lambda i: (0, 0)),
            pl.BlockSpec((8, EMB), lambda i: (0, 0)),
        ],
        out_specs=out_specs,
        out_shape=out_shapes,
    )(x, agg, W1l, b1l.reshape(1, -1), W2l, b2l.reshape(1, -1), t_next)
    if emit_y:
        return res[0], res[1]
    return res[0], None


def _pool_kernel(x, batch_r, proj_W, proj_b):
    def body(x_ref, b_ref, pw_ref, pb_ref, o_ref, sums, counts):
        i = pl.program_id(0)

        @pl.when(i == 0)
        def _():
            sums[...] = jnp.zeros_like(sums)
            counts[...] = jnp.zeros_like(counts)

        gid = lax.broadcasted_iota(jnp.int32, (N_GRAPHS, ROW_BLK), 0)
        oh = (jnp.broadcast_to(b_ref[0], (N_GRAPHS, ROW_BLK)) == gid
              ).astype(F32)
        sums[...] += jnp.dot(oh, x_ref[...], preferred_element_type=F32,
                             precision=HI)
        counts[...] += jnp.broadcast_to(
            jnp.sum(oh, axis=1, keepdims=True), (N_GRAPHS, EMB))

        @pl.when(i == N_BLKS - 1)
        def _():
            pooled = sums[...] / jnp.maximum(counts[...], 1.0)
            o_ref[...] = (
                jnp.dot(pooled, pw_ref[...], preferred_element_type=F32,
                        precision=HI)
                + pb_ref[...])

    return pl.pallas_call(
        body,
        grid=(N_BLKS,),
        in_specs=[
            pl.BlockSpec((ROW_BLK, EMB), lambda i: (i, 0)),
            pl.BlockSpec((1, 1, ROW_BLK), lambda i: (i, 0, 0)),
            pl.BlockSpec((EMB, OUT), lambda i: (0, 0)),
            pl.BlockSpec((1, OUT), lambda i: (0, 0)),
        ],
        out_specs=pl.BlockSpec((N_GRAPHS, OUT), lambda i: (0, 0)),
        out_shape=jax.ShapeDtypeStruct((N_GRAPHS, OUT), F32),
        scratch_shapes=[
            pltpu.VMEM((N_GRAPHS, EMB), F32),
            pltpu.VMEM((N_GRAPHS, EMB), F32),
        ],
    )(x, batch_r, proj_W, proj_b.reshape(1, -1))


def kernel(x_nodes, edge_index, degrees, identifiers, edge_features, batch,
           atom_emb, id_emb, bond_emb, W1, b1, W2, b2, proj_W, proj_b):
    del degrees

    base = atom_emb[:, 0, :].sum(axis=0).reshape(1, EMB)
    diff = atom_emb[:, 1, :] - atom_emb[:, 0, :]
    d_pad = jnp.zeros((16, EMB), F32).at[:9].set(diff)
    xn_pad = jnp.zeros((N_NODES, 16), F32).at[:, :9].set(
        x_nodes.astype(F32))

    bits = jnp.array([[c & 1, (c >> 1) & 1, (c >> 2) & 1] for c in range(8)],
                     dtype=jnp.int32)
    t_combo = (bond_emb[:, 0, bits[:, 0], :]
               + bond_emb[:, 1, bits[:, 1], :]
               + bond_emb[:, 2, bits[:, 2], :])

    combo = (edge_features[:, 0] + 2 * edge_features[:, 1]
             + 4 * edge_features[:, 2]).astype(jnp.int32)
    src = edge_index[0]
    dst = edge_index[1]

    t0 = (id_emb[:, None, :] + t_combo[0][None, :, :]).reshape(
        ID_VOCAB * 8, EMB)
    cidx0 = identifiers[:, 0] * 8 + combo
    yidx = combo * N_NODES + src

    batch_r = batch.reshape(N_BLKS, 1, ROW_BLK)

    sc_l0 = _edge_layer0_call()
    sc_gs = _edge_gather_scatter_call()

    x = _h_kernel(xn_pad, d_pad, base)

    agg = sc_l0(x, t0, src, cidx0, dst)
    x, y = _mlp_kernel(x, agg, W1[0], b1[0], W2[0], b2[0], t_combo[1],
                       relu_out=True, emit_y=True)

    for l in range(1, N_LAYERS):
        agg = sc_gs(y.reshape(8 * N_NODES, EMB), yidx, dst)
        last = l == N_LAYERS - 1
        t_next = t_combo[l + 1] if not last else t_combo[0]
        x, y = _mlp_kernel(x, agg, W1[l], b1[l], W2[l], b2[l], t_next,
                           relu_out=not last, emit_y=not last)

    return _pool_kernel(x, batch_r, proj_W, proj_b)
